# baseline (speedup 1.0000x reference)
import functools

import jax
import jax.numpy as jnp
from jax import lax
from jax.experimental import pallas as pl
from jax.experimental.pallas import tpu as pltpu
from jax.experimental.pallas import tpu_sc as plsc

_B = 8
_NC = 100
_NCTX = 10
_D = 512
_SUF = 66
_TKN = 77
_NW = 32


def _mlp_body(im_ref, ctx_ref, w1_ref, b1_ref, w2_ref, b2_ref, out_ref):
    h = jnp.maximum(
        jnp.dot(im_ref[:], w1_ref[:], preferred_element_type=jnp.float32)
        + b1_ref[:], 0.0)
    bias = jnp.dot(h, w2_ref[:], preferred_element_type=jnp.float32) + b2_ref[:]
    out_ref[:] = ctx_ref[:][None, :, :] + bias[:, None, :]


_sc_mesh = plsc.VectorSubcoreMesh(
    core_axis_name="c", subcore_axis_name="s", num_cores=2, num_subcores=16)


@functools.partial(
    pl.kernel,
    out_type=jax.ShapeDtypeStruct((_B, _NC, _TKN, _D), jnp.float32),
    mesh=_sc_mesh,
    scratch_types=[
        pltpu.VMEM((_B, _NCTX, _D), jnp.float32),
        pltpu.VMEM((1, _D), jnp.float32),
        pltpu.VMEM((_SUF, _D), jnp.float32),
        pltpu.SemaphoreType.DMA,
    ],
    compiler_params=pltpu.CompilerParams(use_tc_tiling_on_sc=False),
)
def _sc_assemble(ctx_hbm, pre_hbm, suf_hbm, out_hbm, ctx_v, pre_v, suf_v, sem):
    wid = lax.axis_index("s") * 2 + lax.axis_index("c")
    pltpu.sync_copy(ctx_hbm, ctx_v)
    for k in range(4):
        c = wid + _NW * k

        @pl.when(c < _NC)
        def _():
            pltpu.sync_copy(pre_hbm.at[c], pre_v)
            pltpu.sync_copy(suf_hbm.at[c], suf_v)
            copies = []
            for b in range(_B):
                copies.append(pltpu.async_copy(
                    pre_v, out_hbm.at[b, c, pl.ds(0, 1)], sem))
                copies.append(pltpu.async_copy(
                    ctx_v.at[b], out_hbm.at[b, c, pl.ds(1, _NCTX)], sem))
                copies.append(pltpu.async_copy(
                    suf_v, out_hbm.at[b, c, pl.ds(1 + _NCTX, _SUF)], sem))
            for cp in copies:
                cp.wait()


def kernel(im_features, ctx, token_prefix, token_suffix, W1, b1, W2, b2):
    ctx_all = pl.pallas_call(
        _mlp_body,
        out_shape=jax.ShapeDtypeStruct((_B, _NCTX, _D), jnp.float32),
    )(im_features, ctx, W1, b1.reshape(1, -1), W2, b2.reshape(1, -1))
    return _sc_assemble(ctx_all, token_prefix, token_suffix)


# speedup vs baseline: 1.3051x; 1.3051x over previous
import functools

import jax
import jax.numpy as jnp
from jax import lax
from jax.experimental import pallas as pl
from jax.experimental.pallas import tpu as pltpu
from jax.experimental.pallas import tpu_sc as plsc

_B = 8
_NC = 100
_NCTX = 10
_D = 512
_SUF = 66
_TKN = 77
_HEAD = 16
_TAIL = _TKN - _HEAD
_HSUF = _HEAD - 1 - _NCTX
_NW = 32
_LC = _D // 16
_STG = 8 + _TAIL


def _mlp_body(im_ref, ctx_ref, w1_ref, b1_ref, w2_ref, b2_ref, out_ref):
    h = jnp.maximum(
        jnp.dot(im_ref[:], w1_ref[:], preferred_element_type=jnp.float32)
        + b1_ref[:], 0.0)
    bias = jnp.dot(h, w2_ref[:], preferred_element_type=jnp.float32) + b2_ref[:]
    out_ref[:] = ctx_ref[:][None, :, :] + bias[:, None, :]


_sc_mesh = plsc.VectorSubcoreMesh(
    core_axis_name="c", subcore_axis_name="s", num_cores=2, num_subcores=16)


@functools.partial(
    pl.kernel,
    out_type=jax.ShapeDtypeStruct((_B, _NC, _TKN, _D), jnp.float32),
    mesh=_sc_mesh,
    scratch_types=[
        pltpu.VMEM((_B, _HEAD, _D), jnp.float32),
        pltpu.VMEM((_STG, _D), jnp.float32),
        pltpu.VMEM((2, _D), jnp.float32),
        pltpu.VMEM((_NCTX, _D), jnp.float32),
        pltpu.VMEM((1, _D), jnp.float32),
        pltpu.SemaphoreType.DMA,
        pltpu.SemaphoreType.DMA,
    ],
)
def _sc_assemble(ctx_hbm, pre_hbm, suf_hbm, out_hbm,
                 head_v, stg_v, suf2_v, tmp_v, pre_v, sem_t, sem_h):
    wid = lax.axis_index("s") * 2 + lax.axis_index("c")

    for b in range(_B):
        pltpu.sync_copy(ctx_hbm.at[b], tmp_v)

        def _pf(r, carry, b=b):
            for l in range(_LC):
                head_v[b, 1 + r, pl.ds(16 * l, 16)] = tmp_v[r, pl.ds(16 * l, 16)]
            return carry

        lax.fori_loop(0, _NCTX, _pf, 0)

    def _class_step(c):
        pltpu.sync_copy(suf_hbm.at[c, pl.ds(0, 64)], stg_v.at[pl.ds(0, 64)])
        pltpu.sync_copy(suf_hbm.at[c, pl.ds(64, 2)], suf2_v)
        pltpu.sync_copy(pre_hbm.at[c], pre_v)

        def _s2(r, carry):
            for l in range(_LC):
                stg_v[64 + r, pl.ds(16 * l, 16)] = suf2_v[r, pl.ds(16 * l, 16)]
            return carry

        lax.fori_loop(0, 2, _s2, 0)

        def _hs(i, carry):
            b = i // _HSUF
            r = i % _HSUF
            for l in range(_LC):
                head_v[b, 1 + _NCTX + r, pl.ds(16 * l, 16)] = \
                    stg_v[r, pl.ds(16 * l, 16)]
            return carry

        lax.fori_loop(0, _B * _HSUF, _hs, 0)

        def _sh(m, carry):
            j = _TAIL - 1 - m
            for l in range(_LC):
                stg_v[8 + j, pl.ds(16 * l, 16)] = stg_v[_HSUF + j, pl.ds(16 * l, 16)]
            return carry

        lax.fori_loop(0, _TAIL, _sh, 0)

        tails = [pltpu.async_copy(
            stg_v.at[pl.ds(8, _TAIL)],
            out_hbm.at[b, c, pl.ds(_HEAD, _TAIL)], sem_t) for b in range(_B)]

        def _hp(b, carry):
            for l in range(_LC):
                head_v[b, 0, pl.ds(16 * l, 16)] = pre_v[0, pl.ds(16 * l, 16)]
            return carry

        lax.fori_loop(0, _B, _hp, 0)

        heads = [pltpu.async_copy(
            head_v.at[b], out_hbm.at[b, c, pl.ds(0, _HEAD)], sem_h)
            for b in range(_B)]
        return tails, heads

    pend_t, pend_h = [], []
    for k in range(3):
        for cp in pend_t:
            cp.wait()
        for cp in pend_h:
            cp.wait()
        pend_t, pend_h = _class_step(wid + _NW * k)
    for cp in pend_t:
        cp.wait()
    for cp in pend_h:
        cp.wait()

    @pl.when(wid + _NW * 3 < _NC)
    def _():
        tails, heads = _class_step(wid + _NW * 3)
        for cp in tails:
            cp.wait()
        for cp in heads:
            cp.wait()


def kernel(im_features, ctx, token_prefix, token_suffix, W1, b1, W2, b2):
    ctx_all = pl.pallas_call(
        _mlp_body,
        out_shape=jax.ShapeDtypeStruct((_B, _NCTX, _D), jnp.float32),
    )(im_features, ctx, W1, b1.reshape(1, -1), W2, b2.reshape(1, -1))
    return _sc_assemble(ctx_all, token_prefix, token_suffix)


# speedup vs baseline: 4.9767x; 3.8132x over previous
import jax
import jax.numpy as jnp
from jax.experimental import pallas as pl
from jax.experimental.pallas import tpu as pltpu

_B = 8
_NC = 100
_NCTX = 10
_D = 512
_SUF = 66
_TKN = 77
_CT = 4


def _body(im_ref, ctx_ref, pre_ref, suf_ref, w1_ref, b1_ref, w2_ref, b2_ref,
          out_ref, ctxp_ref):
    @pl.when(pl.program_id(0) == 0)
    def _():
        h = jnp.maximum(
            jnp.dot(im_ref[:], w1_ref[:], preferred_element_type=jnp.float32)
            + b1_ref[:], 0.0)
        bias = jnp.dot(h, w2_ref[:], preferred_element_type=jnp.float32) + b2_ref[:]
        ctxp_ref[:] = (ctx_ref[:][:, None, :] + bias[None, :, :]).reshape(
            _NCTX * _B, _D)

    out_ref[:, 0:_B, :] = jnp.broadcast_to(
        pre_ref[:].reshape(_CT, 1, _D), (_CT, _B, _D))
    out_ref[:, _B:_B * (1 + _NCTX), :] = jnp.broadcast_to(
        ctxp_ref[:][None], (_CT, _NCTX * _B, _D))
    suf = suf_ref[:]
    out_ref[:, _B * (1 + _NCTX):, :] = jnp.broadcast_to(
        suf[:, :, None, :], (_CT, _SUF, _B, _D)).reshape(_CT, _SUF * _B, _D)


def kernel(im_features, ctx, token_prefix, token_suffix, W1, b1, W2, b2):
    out_p = pl.pallas_call(
        _body,
        grid=(_NC // _CT,),
        in_specs=[
            pl.BlockSpec((_B, _D), lambda c: (0, 0)),
            pl.BlockSpec((_NCTX, _D), lambda c: (0, 0)),
            pl.BlockSpec((_CT, 1, _D), lambda c: (c, 0, 0)),
            pl.BlockSpec((_CT, _SUF, _D), lambda c: (c, 0, 0)),
            pl.BlockSpec((_D, _D // 4), lambda c: (0, 0)),
            pl.BlockSpec((1, _D // 4), lambda c: (0, 0)),
            pl.BlockSpec((_D // 4, _D), lambda c: (0, 0)),
            pl.BlockSpec((1, _D), lambda c: (0, 0)),
        ],
        out_specs=pl.BlockSpec((_CT, _TKN * _B, _D), lambda c: (c, 0, 0)),
        out_shape=jax.ShapeDtypeStruct((_NC, _TKN * _B, _D), jnp.float32),
        scratch_shapes=[pltpu.VMEM((_NCTX * _B, _D), jnp.float32)],
    )(im_features, ctx, token_prefix, token_suffix, W1,
      b1.reshape(1, -1), W2, b2.reshape(1, -1))
    return out_p.reshape(_NC, _TKN, _B, _D).transpose(2, 0, 1, 3)
